# Initial kernel scaffold; baseline (speedup 1.0000x reference)
import jax
import jax.numpy as jnp
from jax.experimental import pallas as pl


def kernel(x, W1, b1, gamma1, beta1, W2, b2, gamma2, beta2, edge_index):
    raise NotImplementedError("write your pallas kernel here")


# speedup vs baseline: 4.4169x; 4.4169x over previous
import functools

import jax
import jax.numpy as jnp
from jax import lax
from jax.experimental import pallas as pl
from jax.experimental.pallas import tpu as pltpu, tpu_sc as plsc

N = 10000
NPAD = 10240
IN_D = 128
HID = 128
OUT_D = 64
E = 320000
NC = 2
NS = 16
NW = NC * NS
K = 128
EPW = ((E + NW * K - 1) // (NW * K)) * K
EPAD = EPW * NW
EPT = EPAD // NS
CHUNKS = EPT // K
NPADH = NPAD // NC
NACC = NPADH + 8
RPT = NPADH // NS
LH = 16

_mesh = lambda: plsc.VectorSubcoreMesh(
    core_axis_name="c", subcore_axis_name="s", num_cores=NC, num_subcores=NS)


@functools.cache
def _make_hist():
    @functools.partial(
        pl.kernel,
        out_type=jax.ShapeDtypeStruct((2, NW, NPAD), jnp.float32),
        mesh=_mesh(),
        compiler_params=pltpu.CompilerParams(needs_layout_passes=False),
        scratch_types=[
            pltpu.VMEM((EPW,), jnp.int32),
            pltpu.VMEM((NPAD,), jnp.float32),
        ],
    )
    def _hist_sc(src_hbm, dst_hbm, zh_hbm, out_hbm, idx_v, hist_v):
        cid = lax.axis_index("c")
        sid = lax.axis_index("s")
        wid = sid * NC + cid
        base = pl.multiple_of(wid * EPW, K)
        ones = jnp.full((16,), 1.0, jnp.float32)

        def count(j, carry):
            v = idx_v[pl.ds(j * 16, 16)]
            plsc.addupdate_scatter(hist_v, [v], ones)
            return carry

        pltpu.sync_copy(zh_hbm, hist_v)
        pltpu.sync_copy(src_hbm.at[pl.ds(base, EPW)], idx_v)
        lax.fori_loop(0, EPW // 16, count, 0)
        pltpu.sync_copy(hist_v, out_hbm.at[0, wid])

        pltpu.sync_copy(zh_hbm, hist_v)
        pltpu.sync_copy(dst_hbm.at[pl.ds(base, EPW)], idx_v)
        lax.fori_loop(0, EPW // 16, count, 0)
        pltpu.sync_copy(hist_v, out_hbm.at[1, wid])

    return _hist_sc


def _make_agg(D):
    @functools.partial(
        pl.kernel,
        out_type=jax.ShapeDtypeStruct((NC, NPADH, D), jnp.float32),
        mesh=_mesh(),
        scratch_types=[
            pltpu.VMEM((K,), jnp.int32),
            pltpu.VMEM((K,), jnp.int32),
            pltpu.VMEM((K,), jnp.int32),
            pltpu.VMEM((K, D), jnp.float32),
            pltpu.VMEM((RPT, D), jnp.float32),
            pltpu.SemaphoreType.DMA,
            pltpu.VMEM_SHARED((NACC, D), jnp.float32),
        ],
    )
    def _agg_sc(z_hbm, src_hbm, dst_hbm, zd_hbm, out_hbm,
                si, di, di2, rows, buf, sem, acc):
        cid = lax.axis_index("c")
        sid = lax.axis_index("s")
        r0 = sid * RPT
        lo = cid * NPADH
        pltpu.sync_copy(zd_hbm.at[pl.ds(r0, RPT)], buf)
        pltpu.sync_copy(buf, acc.at[pl.ds(r0, RPT)])
        plsc.subcore_barrier()

        def chunk(i, carry):
            base = pl.multiple_of(sid * EPT + i * K, K)
            pltpu.sync_copy(src_hbm.at[pl.ds(base, K)], si)
            pltpu.sync_copy(dst_hbm.at[pl.ds(base, K)], di)
            gather = pltpu.async_copy(z_hbm.at[si], rows, sem)
            for j in range(K // 16):
                dv = di[pl.ds(j * 16, 16)]
                m = (dv >= lo) & (dv < lo + NPADH)
                di2[pl.ds(j * 16, 16)] = jnp.where(m, dv - lo, NPADH)
            gather.wait()
            pltpu.sync_copy(rows, acc.at[di2], add=True)
            return carry

        lax.fori_loop(0, CHUNKS, chunk, 0)
        plsc.subcore_barrier()
        pltpu.sync_copy(acc.at[pl.ds(r0, RPT)], buf)
        pltpu.sync_copy(buf, out_hbm.at[cid, pl.ds(r0, RPT)])

    return _agg_sc


_make_agg = functools.cache(_make_agg)


def _deg_norms(hp):
    degs = jnp.sum(hp[0], axis=0)
    degd = jnp.sum(hp[1], axis=0)
    ns = lax.rsqrt(jnp.maximum(degs, 1.0))
    nd = lax.rsqrt(jnp.maximum(degd, 1.0))
    return ns, nd


def _mm1_body(x_ref, w_ref, o_ref):
    o_ref[...] = jnp.dot(x_ref[...], w_ref[...],
                         preferred_element_type=jnp.float32)


def _scale_body(xw_ref, h_ref, o_ref):
    ns, _ = _deg_norms(h_ref[...])
    o_ref[...] = xw_ref[...] * ns[:, None]


def _mid_body(acc_ref, h_ref, b1_ref, g1_ref, be1_ref, w2_ref, o_ref):
    ns, nd = _deg_norms(h_ref[...])
    a = acc_ref[...]
    h = a * nd[:, None] + b1_ref[...][None, :]
    mask = (lax.broadcasted_iota(jnp.int32, (NPAD, 1), 0) < N).astype(jnp.float32)
    mean = jnp.sum(h * mask, axis=0) / N
    cen = h - mean[None, :]
    var = jnp.sum(cen * cen * mask, axis=0) / N
    hbn = cen * lax.rsqrt(var + 1e-5)[None, :] * g1_ref[...][None, :] + be1_ref[...][None, :]
    hr = jnp.maximum(hbn, 0.0)
    z2 = jnp.dot(hr, w2_ref[...], preferred_element_type=jnp.float32)
    o_ref[...] = z2 * ns[:, None] * mask


def _out_body(acc_ref, h_ref, b2_ref, g2_ref, be2_ref, o_ref):
    _, nd = _deg_norms(h_ref[...])
    a = acc_ref[...][:, :OUT_D]
    h = a * nd[:, None] + b2_ref[...][None, :]
    mask = (lax.broadcasted_iota(jnp.int32, (NPAD, 1), 0) < N).astype(jnp.float32)
    mean = jnp.sum(h * mask, axis=0) / N
    cen = h - mean[None, :]
    var = jnp.sum(cen * cen * mask, axis=0) / N
    hbn = cen * lax.rsqrt(var + 1e-5)[None, :] * g2_ref[...][None, :] + be2_ref[...][None, :]
    hh = hbn[:N, :]
    m = jnp.max(hh, axis=1, keepdims=True)
    ex = jnp.exp(hh - m)
    lse = jnp.log(jnp.sum(ex, axis=1, keepdims=True))
    o_ref[...] = hh - m - lse


def _tc_call(body, out_shape, *args):
    return pl.pallas_call(
        body, out_shape=jax.ShapeDtypeStruct(out_shape, jnp.float32))(*args)


def kernel(x, W1, b1, gamma1, beta1, W2, b2, gamma2, beta2, edge_index):
    src = edge_index[0].astype(jnp.int32)
    dst = edge_index[1].astype(jnp.int32)
    pad_e = EPAD - E
    srcp = jnp.concatenate([src, jnp.full((pad_e,), N, jnp.int32)])
    dstp = jnp.concatenate([dst, jnp.full((pad_e,), N, jnp.int32)])
    xp = jnp.pad(x, ((0, NPAD - N), (0, 0)))
    zh = jnp.zeros((NPAD,), jnp.float32)
    z128 = jnp.zeros((NPAD, HID), jnp.float32)
    W2p = jnp.pad(W2, ((0, 0), (0, HID - OUT_D)))

    histp = _make_hist()(srcp, dstp, zh)
    xw1 = _tc_call(_mm1_body, (NPAD, HID), xp, W1)
    z1 = _tc_call(_scale_body, (NPAD, HID), xw1, histp)
    agg = _make_agg(HID)
    acc1 = agg(z1, srcp, dstp, z128)
    z2 = _tc_call(_mid_body, (NPAD, HID), acc1.reshape(NPAD, HID), histp, b1,
                  gamma1, beta1, W2p)
    acc2 = agg(z2, srcp, dstp, z128)
    out = _tc_call(_out_body, (N, OUT_D), acc2.reshape(NPAD, HID), histp, b2,
                  gamma2, beta2)
    return out
